# Initial kernel scaffold; baseline (speedup 1.0000x reference)
import jax
import jax.numpy as jnp
from jax.experimental import pallas as pl


def kernel(types, cont, table, Wc, bc, Wo, bo, gamma, beta):
    raise NotImplementedError("write your pallas kernel here")


# speedup vs baseline: 1.1613x; 1.1613x over previous
import functools

import jax
import jax.numpy as jnp
from jax import lax
from jax.experimental import pallas as pl
from jax.experimental.pallas import tpu as pltpu
from jax.experimental.pallas import tpu_sc as plsc

NUM_ACTIONS = 100000
D_MODEL = 128
HALF = D_MODEL // 2
B = 4096
L = 200
TOKENS = B * L

NW = 32
CHUNK = 128
ROWS_PER_W = TOKENS // NW
CHUNKS = ROWS_PER_W // CHUNK


def _sc_gather(types32, table):
    mesh = plsc.VectorSubcoreMesh(core_axis_name="c", subcore_axis_name="s")

    @functools.partial(
        pl.kernel,
        out_type=jax.ShapeDtypeStruct((TOKENS, HALF), jnp.float32),
        mesh=mesh,
        scratch_types=[
            pltpu.VMEM((CHUNKS, CHUNK), jnp.int32),
            pltpu.VMEM((CHUNK, HALF), jnp.float32),
            pltpu.SemaphoreType.DMA,
        ],
        compiler_params=pltpu.CompilerParams(use_tc_tiling_on_sc=False),
    )
    def gather_kernel(idx_hbm, table_hbm, out_hbm, idx_v, rows_v, sem):
        wid = lax.axis_index("s") * 2 + lax.axis_index("c")
        base = wid * ROWS_PER_W
        pltpu.sync_copy(idx_hbm.at[wid], idx_v)

        def body(j, _):
            pltpu.async_copy(table_hbm.at[idx_v.at[j]], rows_v, sem).wait()
            pltpu.sync_copy(rows_v, out_hbm.at[pl.ds(base + j * CHUNK, CHUNK)])
            return 0

        lax.fori_loop(0, CHUNKS, body, 0)

    return gather_kernel(types32, table)


def _tc_tail(e, cont2d, Wc, bc2, Wo, bo2, gamma2, beta2):
    TBLK = 2048
    grid = (TOKENS // TBLK,)

    def body(e_ref, c_ref, wc_ref, bc_ref, wo_ref, bo_ref, g_ref, b_ref, o_ref):
        wo = wo_ref[...]
        wo_top = wo[:HALF, :]
        wo_bot = wo[HALF:, :]
        w2 = jnp.dot(wc_ref[...], wo_bot, preferred_element_type=jnp.float32)
        b2 = (
            jnp.dot(bc_ref[...], wo_bot, preferred_element_type=jnp.float32)
            + bo_ref[...]
        )
        o = (
            jnp.dot(e_ref[...], wo_top, preferred_element_type=jnp.float32)
            + jnp.dot(c_ref[...], w2, preferred_element_type=jnp.float32)
            + b2
        )
        mu = jnp.mean(o, axis=-1, keepdims=True)
        d = o - mu
        var = jnp.mean(d * d, axis=-1, keepdims=True)
        y = d * lax.rsqrt(var + 1e-5)
        o_ref[...] = y * g_ref[...] + b_ref[...]

    def wspec(shape):
        return pl.BlockSpec(shape, lambda i: (0, 0))

    return pl.pallas_call(
        body,
        grid=grid,
        in_specs=[
            pl.BlockSpec((TBLK, HALF), lambda i: (i, 0)),
            pl.BlockSpec((TBLK, 3), lambda i: (i, 0)),
            wspec((3, HALF)),
            wspec((1, HALF)),
            wspec((D_MODEL, D_MODEL)),
            wspec((1, D_MODEL)),
            wspec((1, D_MODEL)),
            wspec((1, D_MODEL)),
        ],
        out_specs=pl.BlockSpec((TBLK, D_MODEL), lambda i: (i, 0)),
        out_shape=jax.ShapeDtypeStruct((TOKENS, D_MODEL), jnp.float32),
    )(e, cont2d, Wc, bc2, Wo, bo2, gamma2, beta2)


def kernel(types, cont, table, Wc, bc, Wo, bo, gamma, beta):
    types32 = types.astype(jnp.int32).reshape(NW, CHUNKS, CHUNK)
    e = _sc_gather(types32, table)
    cont2d = cont.reshape(TOKENS, 3)
    out = _tc_tail(
        e,
        cont2d,
        Wc,
        bc.reshape(1, HALF),
        Wo,
        bo.reshape(1, D_MODEL),
        gamma.reshape(1, D_MODEL),
        beta.reshape(1, D_MODEL),
    )
    return out.reshape(B, L, D_MODEL)
